# baseline (speedup 1.0000x reference)
import jax
import jax.numpy as jnp
from jax import lax
from jax.experimental import pallas as pl
from jax.experimental.pallas import tpu as pltpu
from jax.experimental.pallas import tpu_sc as plsc

N = 10000
E = 320000
D = 128

NC = 2
NS = 16
NW = NC * NS
EPW = E // NW
C = 80
G = EPW // C
S = 4
NSUP = (G - 1) // S
NPAD = 10112
RPT = NPAD // NS
LPR = D // 16


def _sc_edge_body(idx_hbm, x_hbm, ef_hbm, out_hbm,
                  idq0, big0, big1, xv0, xv1, mv0, mv1,
                  sq0, sbig0, sbig1, sef0, sef1, sx0, sx1, ssc0, ssc1,
                  agg_sh):
    big = (big0, big1)
    xv = (xv0, xv1)
    mv = (mv0, mv1)
    sbig = (sbig0, sbig1)
    sef = (sef0, sef1)
    sx = (sx0, sx1)
    ssc = (ssc0, ssc1)

    c = lax.axis_index("c")
    s = lax.axis_index("s")
    wid = c * NS + s
    ibase = wid * EPW

    @plsc.parallel_loop(0, C, 1, unroll=4)
    def _(r):
        for j in range(LPR):
            mv0[r, pl.ds(j * 16, 16)] = jnp.zeros((16,), jnp.float32)

    zbase = s * RPT
    for t in range(RPT // C):
        pltpu.sync_copy(mv0, agg_sh.at[pl.ds(zbase + t * C, C)])
    rem = RPT % C
    if rem:
        pltpu.sync_copy(mv0.at[pl.ds(0, rem)],
                        agg_sh.at[pl.ds(zbase + (RPT // C) * C, rem)])

    plsc.subcore_barrier()

    def start_big(j, B):
        ebase = pl.multiple_of(ibase + (1 + S * j) * C, 8)
        pltpu.async_copy(idx_hbm.at[:, pl.ds(ebase, S * C)], big[B], sbig[B])

    def wait_big(B):
        pltpu.make_async_copy(
            idx_hbm.at[:, pl.ds(0, S * C)], big[B], sbig[B]).wait()

    def start_in(gofs, j, b, sidx_ref):
        ebase = pl.multiple_of(ibase + (1 + S * j + gofs - 1) * C, 8)
        pltpu.async_copy(x_hbm.at[sidx_ref], xv[b], sx[b])
        pltpu.async_copy(ef_hbm.at[pl.ds(ebase, C)], mv[b], sef[b])

    def wait_in(b):
        pltpu.make_async_copy(ef_hbm.at[pl.ds(0, C)], mv[b], sef[b]).wait()
        pltpu.make_async_copy(x_hbm.at[idq0.at[0]], xv[b], sx[b]).wait()

    def compute(b):
        @plsc.parallel_loop(0, C, 1, unroll=8)
        def _(r):
            for j in range(LPR):
                sl = pl.ds(j * 16, 16)
                mv[b][r, sl] = jnp.maximum(mv[b][r, sl] + xv[b][r, sl], 0.0)

    def start_scatter(b, didx_ref):
        pltpu.async_copy(mv[b], agg_sh.at[didx_ref], ssc[b], add=True)

    def wait_scatter(b):
        pltpu.make_async_copy(mv[b], agg_sh.at[idq0.at[1]], ssc[b]).wait()

    pltpu.async_copy(idx_hbm.at[:, pl.ds(pl.multiple_of(ibase, 8), C)],
                     idq0, sq0)
    start_big(0, 0)
    pltpu.make_async_copy(idx_hbm.at[:, pl.ds(0, C)], idq0, sq0).wait()
    pltpu.async_copy(ef_hbm.at[pl.ds(pl.multiple_of(ibase, 8), C)],
                     mv[0], sef[0])
    pltpu.async_copy(x_hbm.at[idq0.at[0]], xv[0], sx[0])

    wait_big(0)
    start_in(1, 0, 1, big[0].at[0, pl.ds(0, C)])
    wait_in(0)
    compute(0)
    start_scatter(0, idq0.at[1])

    def super_chunk(j, carry):
        B = j % 2

        for k in range(S):
            b = (1 + k) % 2
            wait_scatter(b ^ 1)
            if k < S - 1:
                guard0 = j % 2 == 0
                guard1 = j % 2 == 1
            else:
                guard0 = (j % 2 == 0) & (j < NSUP - 1)
                guard1 = (j % 2 == 1) & (j < NSUP - 1)

            @pl.when(guard0)
            def _():
                if k == S - 1:
                    wait_big(1)
                    start_in(k + 2, j, b ^ 1, big[1].at[0, pl.ds(0, C)])
                else:
                    start_in(k + 2, j, b ^ 1,
                             big[0].at[0, pl.ds((k + 1) * C, C)])

            @pl.when(guard1)
            def _():
                if k == S - 1:
                    wait_big(0)
                    start_in(k + 2, j, b ^ 1, big[0].at[0, pl.ds(0, C)])
                else:
                    start_in(k + 2, j, b ^ 1,
                             big[1].at[0, pl.ds((k + 1) * C, C)])
            wait_in(b)
            compute(b)
            @pl.when(j % 2 == 0)
            def _():
                start_scatter(b, big[0].at[1, pl.ds(k * C, C)])

            @pl.when(j % 2 == 1)
            def _():
                start_scatter(b, big[1].at[1, pl.ds(k * C, C)])
            if k == 1:
                @pl.when(j < NSUP - 1)
                def _():
                    @pl.when(j % 2 == 0)
                    def _():
                        start_big(j + 1, 1)

                    @pl.when(j % 2 == 1)
                    def _():
                        start_big(j + 1, 0)
        return carry

    lax.fori_loop(0, NSUP, super_chunk, 0)

    wait_scatter(0)

    plsc.subcore_barrier()

    rbase = s * RPT
    pltpu.sync_copy(agg_sh.at[pl.ds(rbase, RPT)],
                    out_hbm.at[c, pl.ds(rbase, RPT)])


@jax.jit
def _sc_edge_phase(idx2, x, ef):
    mesh = plsc.VectorSubcoreMesh(core_axis_name="c", subcore_axis_name="s")
    k = pl.kernel(
        _sc_edge_body,
        out_type=jax.ShapeDtypeStruct((NC, NPAD, D), jnp.float32),
        mesh=mesh,
        compiler_params=pltpu.CompilerParams(use_tc_tiling_on_sc=False),
        scratch_types=[
            pltpu.VMEM((2, C), jnp.int32),
            pltpu.VMEM((2, S * C), jnp.int32),
            pltpu.VMEM((2, S * C), jnp.int32),
            pltpu.VMEM((C, D), jnp.float32),
            pltpu.VMEM((C, D), jnp.float32),
            pltpu.VMEM((C, D), jnp.float32),
            pltpu.VMEM((C, D), jnp.float32),
            pltpu.SemaphoreType.DMA,
            pltpu.SemaphoreType.DMA,
            pltpu.SemaphoreType.DMA,
            pltpu.SemaphoreType.DMA,
            pltpu.SemaphoreType.DMA,
            pltpu.SemaphoreType.DMA,
            pltpu.SemaphoreType.DMA,
            pltpu.SemaphoreType.DMA,
            pltpu.SemaphoreType.DMA,
            pltpu.VMEM_SHARED((NPAD, D), jnp.float32),
        ],
    )
    return k(idx2, x, ef)


def _mlp_body(x_ref, p_ref, w1_ref, b1_ref, w2_ref, b2_ref, o_ref):
    h = x_ref[...] + p_ref[0] + p_ref[1]
    h = jnp.maximum(
        jnp.dot(h, w1_ref[...], preferred_element_type=jnp.float32) + b1_ref[...],
        0.0)
    o_ref[...] = (
        jnp.dot(h, w2_ref[...], preferred_element_type=jnp.float32) + b2_ref[...])


BR = 2000


@jax.jit
def _mlp_phase(x, p, W1, b1, W2, b2):
    grid = (N // BR,)
    return pl.pallas_call(
        _mlp_body,
        grid=grid,
        in_specs=[
            pl.BlockSpec((BR, D), lambda i: (i, 0)),
            pl.BlockSpec((NC, BR, D), lambda i: (0, i, 0)),
            pl.BlockSpec((D, D), lambda i: (0, 0)),
            pl.BlockSpec((1, D), lambda i: (0, 0)),
            pl.BlockSpec((D, D), lambda i: (0, 0)),
            pl.BlockSpec((1, D), lambda i: (0, 0)),
        ],
        out_specs=pl.BlockSpec((BR, D), lambda i: (i, 0)),
        out_shape=jax.ShapeDtypeStruct((N, D), jnp.float32),
    )(x, p, W1, b1.reshape(1, D), W2, b2.reshape(1, D))


def kernel(x, edge_index, identifiers, degrees, edge_features, W1, b1, W2, b2):
    p = _sc_edge_phase(edge_index, x, edge_features)
    return _mlp_phase(x, p, W1, b1, W2, b2)


# speedup vs baseline: 1.0630x; 1.0630x over previous
import jax
import jax.numpy as jnp
from jax import lax
from jax.experimental import pallas as pl
from jax.experimental.pallas import tpu as pltpu
from jax.experimental.pallas import tpu_sc as plsc

N = 10000
E = 320000
D = 128

NC = 2
NS = 16
NW = NC * NS
EPW = E // NW
C = 80
G = EPW // C
S = 4
NSUP = (G - 1) // S
NPAD = 10112
RPT = NPAD // NS
LPR = D // 16


def _sc_edge_body(idx_hbm, x_hbm, ef_hbm, out_hbm,
                  idq0, big0, big1, xv0, xv1, mv0, mv1,
                  sq0, sbig0, sbig1, sef0, sef1, sx0, sx1, ssc0, ssc1,
                  agg_sh):
    big = (big0, big1)
    xv = (xv0, xv1)
    mv = (mv0, mv1)
    sbig = (sbig0, sbig1)
    sef = (sef0, sef1)
    sx = (sx0, sx1)
    ssc = (ssc0, ssc1)

    c = lax.axis_index("c")
    s = lax.axis_index("s")
    wid = c * NS + s
    ibase = wid * EPW

    @plsc.parallel_loop(0, C, 1, unroll=4)
    def _(r):
        for j in range(LPR):
            mv0[r, pl.ds(j * 16, 16)] = jnp.zeros((16,), jnp.float32)

    zbase = s * RPT
    for t in range(RPT // C):
        pltpu.sync_copy(mv0, agg_sh.at[pl.ds(zbase + t * C, C)])
    rem = RPT % C
    if rem:
        pltpu.sync_copy(mv0.at[pl.ds(0, rem)],
                        agg_sh.at[pl.ds(zbase + (RPT // C) * C, rem)])

    plsc.subcore_barrier()

    def start_big(j, B):
        ebase = pl.multiple_of(ibase + (1 + S * j) * C, 8)
        pltpu.async_copy(idx_hbm.at[:, pl.ds(ebase, S * C)], big[B], sbig[B])

    def wait_big(B):
        pltpu.make_async_copy(
            idx_hbm.at[:, pl.ds(0, S * C)], big[B], sbig[B]).wait()

    def start_in(gofs, j, b, sidx_ref):
        ebase = pl.multiple_of(ibase + (1 + S * j + gofs - 1) * C, 8)
        pltpu.async_copy(x_hbm.at[sidx_ref], xv[b], sx[b])
        pltpu.async_copy(ef_hbm.at[pl.ds(ebase, C)], mv[b], sef[b])

    def wait_in(b):
        pltpu.make_async_copy(ef_hbm.at[pl.ds(0, C)], mv[b], sef[b]).wait()
        pltpu.make_async_copy(x_hbm.at[idq0.at[0]], xv[b], sx[b]).wait()

    def compute(b):
        @plsc.parallel_loop(0, C, 1, unroll=4)
        def _(r):
            for j in range(LPR):
                sl = pl.ds(j * 16, 16)
                mv[b][r, sl] = jnp.maximum(mv[b][r, sl] + xv[b][r, sl], 0.0)

    def start_scatter(b, didx_ref):
        pltpu.async_copy(mv[b], agg_sh.at[didx_ref], ssc[b], add=True)

    def wait_scatter(b):
        pltpu.make_async_copy(mv[b], agg_sh.at[idq0.at[1]], ssc[b]).wait()

    pltpu.async_copy(idx_hbm.at[:, pl.ds(pl.multiple_of(ibase, 8), C)],
                     idq0, sq0)
    start_big(0, 0)
    pltpu.make_async_copy(idx_hbm.at[:, pl.ds(0, C)], idq0, sq0).wait()
    pltpu.async_copy(ef_hbm.at[pl.ds(pl.multiple_of(ibase, 8), C)],
                     mv[0], sef[0])
    pltpu.async_copy(x_hbm.at[idq0.at[0]], xv[0], sx[0])

    wait_big(0)
    start_in(1, 0, 1, big[0].at[0, pl.ds(0, C)])
    wait_in(0)
    compute(0)
    start_scatter(0, idq0.at[1])

    def super_chunk(j, carry):
        B = j % 2

        for k in range(S):
            b = (1 + k) % 2
            wait_scatter(b ^ 1)
            if k < S - 1:
                guard0 = j % 2 == 0
                guard1 = j % 2 == 1
            else:
                guard0 = (j % 2 == 0) & (j < NSUP - 1)
                guard1 = (j % 2 == 1) & (j < NSUP - 1)

            @pl.when(guard0)
            def _():
                if k == S - 1:
                    wait_big(1)
                    start_in(k + 2, j, b ^ 1, big[1].at[0, pl.ds(0, C)])
                else:
                    start_in(k + 2, j, b ^ 1,
                             big[0].at[0, pl.ds((k + 1) * C, C)])

            @pl.when(guard1)
            def _():
                if k == S - 1:
                    wait_big(0)
                    start_in(k + 2, j, b ^ 1, big[0].at[0, pl.ds(0, C)])
                else:
                    start_in(k + 2, j, b ^ 1,
                             big[1].at[0, pl.ds((k + 1) * C, C)])
            wait_in(b)
            compute(b)
            @pl.when(j % 2 == 0)
            def _():
                start_scatter(b, big[0].at[1, pl.ds(k * C, C)])

            @pl.when(j % 2 == 1)
            def _():
                start_scatter(b, big[1].at[1, pl.ds(k * C, C)])
            if k == 1:
                @pl.when(j < NSUP - 1)
                def _():
                    @pl.when(j % 2 == 0)
                    def _():
                        start_big(j + 1, 1)

                    @pl.when(j % 2 == 1)
                    def _():
                        start_big(j + 1, 0)
        return carry

    lax.fori_loop(0, NSUP, super_chunk, 0)

    wait_scatter(0)

    plsc.subcore_barrier()

    rbase = s * RPT
    pltpu.sync_copy(agg_sh.at[pl.ds(rbase, RPT)],
                    out_hbm.at[c, pl.ds(rbase, RPT)])


@jax.jit
def _sc_edge_phase(idx2, x, ef):
    mesh = plsc.VectorSubcoreMesh(core_axis_name="c", subcore_axis_name="s")
    k = pl.kernel(
        _sc_edge_body,
        out_type=jax.ShapeDtypeStruct((NC, NPAD, D), jnp.float32),
        mesh=mesh,
        compiler_params=pltpu.CompilerParams(use_tc_tiling_on_sc=False),
        scratch_types=[
            pltpu.VMEM((2, C), jnp.int32),
            pltpu.VMEM((2, S * C), jnp.int32),
            pltpu.VMEM((2, S * C), jnp.int32),
            pltpu.VMEM((C, D), jnp.float32),
            pltpu.VMEM((C, D), jnp.float32),
            pltpu.VMEM((C, D), jnp.float32),
            pltpu.VMEM((C, D), jnp.float32),
            pltpu.SemaphoreType.DMA,
            pltpu.SemaphoreType.DMA,
            pltpu.SemaphoreType.DMA,
            pltpu.SemaphoreType.DMA,
            pltpu.SemaphoreType.DMA,
            pltpu.SemaphoreType.DMA,
            pltpu.SemaphoreType.DMA,
            pltpu.SemaphoreType.DMA,
            pltpu.SemaphoreType.DMA,
            pltpu.VMEM_SHARED((NPAD, D), jnp.float32),
        ],
    )
    return k(idx2, x, ef)


def _mlp_body(x_ref, p_ref, w1_ref, b1_ref, w2_ref, b2_ref, o_ref):
    h = x_ref[...] + p_ref[0] + p_ref[1]
    h = jnp.maximum(
        jnp.dot(h, w1_ref[...], preferred_element_type=jnp.float32) + b1_ref[...],
        0.0)
    o_ref[...] = (
        jnp.dot(h, w2_ref[...], preferred_element_type=jnp.float32) + b2_ref[...])


BR = 2000


@jax.jit
def _mlp_phase(x, p, W1, b1, W2, b2):
    grid = (N // BR,)
    return pl.pallas_call(
        _mlp_body,
        grid=grid,
        in_specs=[
            pl.BlockSpec((BR, D), lambda i: (i, 0)),
            pl.BlockSpec((NC, BR, D), lambda i: (0, i, 0)),
            pl.BlockSpec((D, D), lambda i: (0, 0)),
            pl.BlockSpec((1, D), lambda i: (0, 0)),
            pl.BlockSpec((D, D), lambda i: (0, 0)),
            pl.BlockSpec((1, D), lambda i: (0, 0)),
        ],
        out_specs=pl.BlockSpec((BR, D), lambda i: (i, 0)),
        out_shape=jax.ShapeDtypeStruct((N, D), jnp.float32),
    )(x, p, W1, b1.reshape(1, D), W2, b2.reshape(1, D))


def kernel(x, edge_index, identifiers, degrees, edge_features, W1, b1, W2, b2):
    p = _sc_edge_phase(edge_index, x, edge_features)
    return _mlp_phase(x, p, W1, b1, W2, b2)
